# Initial kernel scaffold; baseline (speedup 1.0000x reference)
import jax
import jax.numpy as jnp
from jax.experimental import pallas as pl


def kernel(x, W, bias):
    raise NotImplementedError("write your pallas kernel here")


# speedup vs baseline: 1.4181x; 1.4181x over previous
import functools

import jax
import jax.numpy as jnp
from jax import lax
from jax.experimental import pallas as pl
from jax.experimental.pallas import tpu as pltpu
from jax.experimental.pallas import tpu_sc as plsc

BATCH = 16384
FIELDS = 26
NUM_CORES = 2
NUM_SUBCORES = 16
NW = NUM_CORES * NUM_SUBCORES
BPW = BATCH // NW
IPW = BPW * FIELDS
LANES = 16
CHUNKS = BPW // LANES


def _sc_kernel(idx_hbm, w_hbm, bias_hbm, out_hbm, idx_v, vals_v, out_v,
               bias_v, sem):
    c = lax.axis_index("c")
    s = lax.axis_index("s")
    wid = s * NUM_CORES + c
    base = wid * IPW

    pltpu.sync_copy(idx_hbm.at[pl.ds(base, IPW)], idx_v)
    pltpu.sync_copy(bias_hbm, bias_v)
    pltpu.async_copy(w_hbm.at[idx_v], vals_v, sem).wait()

    bvec = bias_v[...]
    for ci in range(CHUNKS):
        acc = bvec
        for f in range(FIELDS):
            acc = acc + vals_v[pl.ds(f * BPW + ci * LANES, LANES)]
        out_v[pl.ds(ci * LANES, LANES)] = acc

    pltpu.sync_copy(out_v, out_hbm.at[pl.ds(wid * BPW, BPW)])


@jax.jit
def kernel(x, W, bias):
    idx = x.reshape(NW, BPW, FIELDS).transpose(0, 2, 1).reshape(NW * IPW)
    w_flat = W.reshape(-1)
    bias16 = jnp.broadcast_to(bias, (LANES,))

    mesh = plsc.VectorSubcoreMesh(core_axis_name="c", subcore_axis_name="s")
    run = functools.partial(
        pl.kernel,
        mesh=mesh,
        out_type=jax.ShapeDtypeStruct((BATCH,), jnp.float32),
        scratch_types=[
            pltpu.VMEM((IPW,), jnp.int32),
            pltpu.VMEM((IPW,), jnp.float32),
            pltpu.VMEM((BPW,), jnp.float32),
            pltpu.VMEM((LANES,), jnp.float32),
            pltpu.SemaphoreType.DMA,
        ],
    )(_sc_kernel)
    out = run(idx, w_flat, bias16)
    return out.reshape(BATCH, 1)


# speedup vs baseline: 2.8531x; 2.0119x over previous
import jax
import jax.numpy as jnp
from jax import lax
from jax.experimental import pallas as pl
from jax.experimental.pallas import tpu as pltpu
from jax.experimental.pallas import tpu_sc as plsc

BATCH = 16384
FIELDS = 26
NUM_CORES = 2
NUM_SUBCORES = 16
NW = NUM_CORES * NUM_SUBCORES
BPW = BATCH // NW
IPW = BPW * FIELDS
LANES = 16
CHUNKS = BPW // LANES


def _sc_kernel(idx_hbm, w_hbm, bias_hbm, out_hbm, idx_v, vals_v, out_v,
               bias_v, sem):
    c = lax.axis_index("c")
    s = lax.axis_index("s")
    wid = s * NUM_CORES + c
    base = wid * IPW

    pltpu.sync_copy(idx_hbm.at[pl.ds(base, IPW)], idx_v)
    pltpu.sync_copy(bias_hbm, bias_v)
    w1 = w_hbm.at[0]
    pltpu.async_copy(w1.at[idx_v], vals_v, sem).wait()

    bvec = bias_v[...]
    for ci in range(CHUNKS):
        acc = bvec
        for f in range(FIELDS):
            acc = acc + vals_v[pl.ds(f * BPW + ci * LANES, LANES)]
        out_v[pl.ds(ci * LANES, LANES)] = acc

    pltpu.sync_copy(out_v, out_hbm.at[pl.ds(wid * BPW, BPW)])


@jax.jit
def kernel(x, W, bias):
    idx = x.reshape(NW, BPW, FIELDS).transpose(0, 2, 1).reshape(NW * IPW)
    wT = W.T
    bias16 = jnp.broadcast_to(bias, (LANES,))

    mesh = plsc.VectorSubcoreMesh(core_axis_name="c", subcore_axis_name="s")
    run = pl.kernel(
        _sc_kernel,
        mesh=mesh,
        out_type=jax.ShapeDtypeStruct((BATCH,), jnp.float32),
        scratch_types=[
            pltpu.VMEM((IPW,), jnp.int32),
            pltpu.VMEM((IPW,), jnp.float32),
            pltpu.VMEM((BPW,), jnp.float32),
            pltpu.VMEM((LANES,), jnp.float32),
            pltpu.SemaphoreType.DMA,
        ],
    )
    return run(idx, wT, bias16).reshape(BATCH, 1)


# speedup vs baseline: 2.9244x; 1.0250x over previous
import jax
import jax.numpy as jnp
from jax import lax
from jax.experimental import pallas as pl
from jax.experimental.pallas import tpu as pltpu
from jax.experimental.pallas import tpu_sc as plsc

BATCH = 16384
FIELDS = 26
NUM_CORES = 2
NUM_SUBCORES = 16
NW = NUM_CORES * NUM_SUBCORES
BPW = BATCH // NW
IPW = BPW * FIELDS
LANES = 16
CHUNKS = BPW // LANES


def _sc_kernel(idx_hbm, w_hbm, bias_hbm, out_hbm, idx_v, vals_v, out_v,
               bias_v, sem):
    c = lax.axis_index("c")
    s = lax.axis_index("s")
    wid = s * NUM_CORES + c
    base = wid * IPW

    pltpu.sync_copy(idx_hbm.at[pl.ds(base, IPW)], idx_v)
    pltpu.sync_copy(bias_hbm, bias_v)
    w1 = w_hbm.at[0]
    pltpu.async_copy(w1.at[idx_v], vals_v, sem).wait()

    bvec = bias_v[...]

    def chunk_body(ci, _):
        off = ci * LANES
        acc = bvec
        for f in range(FIELDS):
            acc = acc + vals_v[pl.ds(f * BPW + off, LANES)]
        out_v[pl.ds(off, LANES)] = acc
        return 0

    lax.fori_loop(0, CHUNKS, chunk_body, 0, unroll=False)

    pltpu.sync_copy(out_v, out_hbm.at[pl.ds(wid * BPW, BPW)])


@jax.jit
def kernel(x, W, bias):
    idx = x.reshape(NW, BPW, FIELDS).transpose(0, 2, 1).reshape(NW * IPW)
    wT = W.T
    bias16 = jnp.broadcast_to(bias, (LANES,))

    mesh = plsc.VectorSubcoreMesh(core_axis_name="c", subcore_axis_name="s")
    run = pl.kernel(
        _sc_kernel,
        mesh=mesh,
        out_type=jax.ShapeDtypeStruct((BATCH,), jnp.float32),
        scratch_types=[
            pltpu.VMEM((IPW,), jnp.int32),
            pltpu.VMEM((IPW,), jnp.float32),
            pltpu.VMEM((BPW,), jnp.float32),
            pltpu.VMEM((LANES,), jnp.float32),
            pltpu.SemaphoreType.DMA,
        ],
    )
    return run(idx, wT, bias16).reshape(BATCH, 1)


# speedup vs baseline: 3.0352x; 1.0379x over previous
import jax
import jax.numpy as jnp
from jax import lax
from jax.experimental import pallas as pl
from jax.experimental.pallas import tpu as pltpu
from jax.experimental.pallas import tpu_sc as plsc

BATCH = 16384
FIELDS = 26
NUM_CORES = 2
NUM_SUBCORES = 16
NW = NUM_CORES * NUM_SUBCORES
BPW = BATCH // NW
IPW = BPW * FIELDS
LANES = 16
HALVES = 2
BPH = BPW // HALVES
IPH = BPH * FIELDS
CHUNKS_H = BPH // LANES


def _sc_kernel(idx_hbm, w_hbm, bias_hbm, out_hbm, idx_v, vals_v, out_v,
               bias_v, sem0, sem1):
    c = lax.axis_index("c")
    s = lax.axis_index("s")
    wid = s * NUM_CORES + c
    base = wid * IPW
    w1 = w_hbm.at[0]
    sems = (sem0, sem1)

    gathers = []
    for h in range(HALVES):
        pltpu.sync_copy(idx_hbm.at[pl.ds(base + h * IPH, IPH)],
                        idx_v.at[pl.ds(h * IPH, IPH)])
        gathers.append(
            pltpu.async_copy(w1.at[idx_v.at[pl.ds(h * IPH, IPH)]],
                             vals_v.at[pl.ds(h * IPH, IPH)], sems[h]))
    pltpu.sync_copy(bias_hbm, bias_v)
    bvec = bias_v[...]

    for h in range(HALVES):
        gathers[h].wait()

        def chunk_body(ci, _, h=h):
            off = ci * LANES
            acc = bvec
            for f in range(FIELDS):
                acc = acc + vals_v[pl.ds(h * IPH + f * BPH + off, LANES)]
            out_v[pl.ds(h * BPH + off, LANES)] = acc
            return 0

        lax.fori_loop(0, CHUNKS_H, chunk_body, 0, unroll=False)

    pltpu.sync_copy(out_v, out_hbm.at[pl.ds(wid * BPW, BPW)])


@jax.jit
def kernel(x, W, bias):
    idx = (x.reshape(NW, HALVES, BPH, FIELDS)
           .transpose(0, 1, 3, 2)
           .reshape(NW * IPW))
    wT = W.T
    bias16 = jnp.broadcast_to(bias, (LANES,))

    mesh = plsc.VectorSubcoreMesh(core_axis_name="c", subcore_axis_name="s")
    run = pl.kernel(
        _sc_kernel,
        mesh=mesh,
        out_type=jax.ShapeDtypeStruct((BATCH,), jnp.float32),
        scratch_types=[
            pltpu.VMEM((IPW,), jnp.int32),
            pltpu.VMEM((IPW,), jnp.float32),
            pltpu.VMEM((BPW,), jnp.float32),
            pltpu.VMEM((LANES,), jnp.float32),
            pltpu.SemaphoreType.DMA,
            pltpu.SemaphoreType.DMA,
        ],
    )
    return run(idx, wT, bias16).reshape(BATCH, 1)


# speedup vs baseline: 3.0675x; 1.0106x over previous
import jax
import jax.numpy as jnp
from jax import lax
from jax.experimental import pallas as pl
from jax.experimental.pallas import tpu as pltpu
from jax.experimental.pallas import tpu_sc as plsc

BATCH = 16384
FIELDS = 26
NUM_CORES = 2
NUM_SUBCORES = 16
NW = NUM_CORES * NUM_SUBCORES
BPW = BATCH // NW
IPW = BPW * FIELDS
LANES = 16
HALVES = 4
BPH = BPW // HALVES
IPH = BPH * FIELDS
CHUNKS_H = BPH // LANES


def _sc_kernel(idx_hbm, w_hbm, bias_hbm, out_hbm, idx_v, vals_v, out_v,
               bias_v, sem0, sem1, sem2, sem3):
    c = lax.axis_index("c")
    s = lax.axis_index("s")
    wid = s * NUM_CORES + c
    base = wid * IPW
    w1 = w_hbm.at[0]
    sems = (sem0, sem1, sem2, sem3)

    gathers = []
    for h in range(HALVES):
        pltpu.sync_copy(idx_hbm.at[pl.ds(base + h * IPH, IPH)],
                        idx_v.at[pl.ds(h * IPH, IPH)])
        gathers.append(
            pltpu.async_copy(w1.at[idx_v.at[pl.ds(h * IPH, IPH)]],
                             vals_v.at[pl.ds(h * IPH, IPH)], sems[h]))
    pltpu.sync_copy(bias_hbm, bias_v)
    bvec = bias_v[...]

    for h in range(HALVES):
        gathers[h].wait()

        def chunk_body(ci, _, h=h):
            off = ci * LANES
            acc = bvec
            for f in range(FIELDS):
                acc = acc + vals_v[pl.ds(h * IPH + f * BPH + off, LANES)]
            out_v[pl.ds(h * BPH + off, LANES)] = acc
            return 0

        lax.fori_loop(0, CHUNKS_H, chunk_body, 0, unroll=False)

    pltpu.sync_copy(out_v, out_hbm.at[pl.ds(wid * BPW, BPW)])


@jax.jit
def kernel(x, W, bias):
    idx = (x.reshape(NW, HALVES, BPH, FIELDS)
           .transpose(0, 1, 3, 2)
           .reshape(NW * IPW))
    wT = W.T
    bias16 = jnp.broadcast_to(bias, (LANES,))

    mesh = plsc.VectorSubcoreMesh(core_axis_name="c", subcore_axis_name="s")
    run = pl.kernel(
        _sc_kernel,
        mesh=mesh,
        out_type=jax.ShapeDtypeStruct((BATCH,), jnp.float32),
        scratch_types=[
            pltpu.VMEM((IPW,), jnp.int32),
            pltpu.VMEM((IPW,), jnp.float32),
            pltpu.VMEM((BPW,), jnp.float32),
            pltpu.VMEM((LANES,), jnp.float32),
            pltpu.SemaphoreType.DMA,
            pltpu.SemaphoreType.DMA,
            pltpu.SemaphoreType.DMA,
            pltpu.SemaphoreType.DMA,
        ],
    )
    return run(idx, wT, bias16).reshape(BATCH, 1)


# speedup vs baseline: 3.1793x; 1.0364x over previous
import jax
import jax.numpy as jnp
from jax import lax
from jax.experimental import pallas as pl
from jax.experimental.pallas import tpu as pltpu
from jax.experimental.pallas import tpu_sc as plsc

BATCH = 16384
FIELDS = 26
NUM_CORES = 2
NUM_SUBCORES = 16
NW = NUM_CORES * NUM_SUBCORES
BPW = BATCH // NW
IPW = BPW * FIELDS
LANES = 16
QUARTERS = 4
BPQ = BPW // QUARTERS
IPQ = BPQ * FIELDS
CHUNKS_Q = BPQ // LANES


def _sc_kernel(xT_hbm, w_hbm, bias_hbm, out_hbm, idx_v, vals_v, out_v,
               bias_v, csem, sem0, sem1, sem2, sem3):
    c = lax.axis_index("c")
    s = lax.axis_index("s")
    wid = s * NUM_CORES + c
    bbase = wid * BPW
    w1 = w_hbm.at[0]
    idx1 = idx_v.at[0]
    sems = (sem0, sem1, sem2, sem3)

    gathers = []
    for q in range(QUARTERS):
        copies = []
        for f in range(FIELDS):
            copies.append(pltpu.async_copy(
                xT_hbm.at[pl.ds(f, 1), pl.ds(bbase + q * BPQ, BPQ)],
                idx_v.at[pl.ds(0, 1), pl.ds(q * IPQ + f * BPQ, BPQ)],
                csem))
        for cp in copies:
            cp.wait()
        gathers.append(
            pltpu.async_copy(w1.at[idx1.at[pl.ds(q * IPQ, IPQ)]],
                             vals_v.at[pl.ds(q * IPQ, IPQ)], sems[q]))
    pltpu.sync_copy(bias_hbm, bias_v)
    bvec = bias_v[...]

    for q in range(QUARTERS):
        gathers[q].wait()

        def chunk_body(ci, _, q=q):
            off = ci * LANES
            acc = bvec
            for f in range(FIELDS):
                acc = acc + vals_v[pl.ds(q * IPQ + f * BPQ + off, LANES)]
            out_v[pl.ds(q * BPQ + off, LANES)] = acc
            return 0

        lax.fori_loop(0, CHUNKS_Q, chunk_body, 0, unroll=False)

    pltpu.sync_copy(out_v, out_hbm.at[pl.ds(bbase, BPW)])


@jax.jit
def kernel(x, W, bias):
    xT = x.T
    wT = W.T
    bias16 = jnp.broadcast_to(bias, (LANES,))

    mesh = plsc.VectorSubcoreMesh(core_axis_name="c", subcore_axis_name="s")
    run = pl.kernel(
        _sc_kernel,
        mesh=mesh,
        out_type=jax.ShapeDtypeStruct((BATCH,), jnp.float32),
        scratch_types=[
            pltpu.VMEM((1, IPW), jnp.int32),
            pltpu.VMEM((IPW,), jnp.float32),
            pltpu.VMEM((BPW,), jnp.float32),
            pltpu.VMEM((LANES,), jnp.float32),
            pltpu.SemaphoreType.DMA,
            pltpu.SemaphoreType.DMA,
            pltpu.SemaphoreType.DMA,
            pltpu.SemaphoreType.DMA,
            pltpu.SemaphoreType.DMA,
        ],
    )
    return run(xT, wT, bias16).reshape(BATCH, 1)


# speedup vs baseline: 3.2094x; 1.0095x over previous
import jax
import jax.numpy as jnp
from jax import lax
from jax.experimental import pallas as pl
from jax.experimental.pallas import tpu as pltpu
from jax.experimental.pallas import tpu_sc as plsc

BATCH = 16384
FIELDS = 26
NUM_CORES = 2
NUM_SUBCORES = 16
NW = NUM_CORES * NUM_SUBCORES
BPW = BATCH // NW
IPW = BPW * FIELDS
LANES = 16
QUARTERS = 4
BPQ = BPW // QUARTERS
IPQ = BPQ * FIELDS
CHUNKS_Q = BPQ // LANES


def _sc_kernel(xT_hbm, w_hbm, bias_hbm, out_hbm, idx_v, vals_v, out_v,
               bias_v, csem, sem0, sem1, sem2, sem3):
    c = lax.axis_index("c")
    s = lax.axis_index("s")
    wid = s * NUM_CORES + c
    bbase = wid * BPW
    w1 = w_hbm.at[0]
    idx1 = idx_v.at[0]
    sems = (sem0, sem1, sem2, sem3)

    gathers = []
    for q in range(QUARTERS):
        copies = []
        for f in range(FIELDS):
            copies.append(pltpu.async_copy(
                xT_hbm.at[pl.ds(f, 1), pl.ds(bbase + q * BPQ, BPQ)],
                idx_v.at[pl.ds(0, 1), pl.ds(q * IPQ + f * BPQ, BPQ)],
                csem))
        for cp in copies:
            cp.wait()
        gathers.append(
            pltpu.async_copy(w1.at[idx1.at[pl.ds(q * IPQ, IPQ)]],
                             vals_v.at[pl.ds(q * IPQ, IPQ)], sems[q]))
    pltpu.sync_copy(bias_hbm, bias_v)
    bvec = bias_v[...]

    for q in range(QUARTERS):
        gathers[q].wait()

        def chunk_body(ci, _, q=q):
            off = ci * LANES
            half = FIELDS // 2
            acc0 = bvec + vals_v[pl.ds(q * IPQ + half * BPQ + off, LANES)]
            acc1 = vals_v[pl.ds(q * IPQ + (half + 1) * BPQ + off, LANES)]
            acc2 = vals_v[pl.ds(q * IPQ + (half + 2) * BPQ + off, LANES)]
            for f in range(half):
                acc0 = acc0 + vals_v[pl.ds(q * IPQ + f * BPQ + off, LANES)]
                if half + 3 + f < FIELDS:
                    acc1 = acc1 + vals_v[
                        pl.ds(q * IPQ + (half + 3 + f) * BPQ + off, LANES)]
            out_v[pl.ds(q * BPQ + off, LANES)] = acc0 + acc1 + acc2
            return 0

        lax.fori_loop(0, CHUNKS_Q, chunk_body, 0, unroll=False)

    pltpu.sync_copy(out_v, out_hbm.at[pl.ds(bbase, BPW)])


@jax.jit
def kernel(x, W, bias):
    xT = x.T
    wT = W.T
    bias16 = jnp.broadcast_to(bias, (LANES,))

    mesh = plsc.VectorSubcoreMesh(core_axis_name="c", subcore_axis_name="s")
    run = pl.kernel(
        _sc_kernel,
        mesh=mesh,
        out_type=jax.ShapeDtypeStruct((BATCH,), jnp.float32),
        scratch_types=[
            pltpu.VMEM((1, IPW), jnp.int32),
            pltpu.VMEM((IPW,), jnp.float32),
            pltpu.VMEM((BPW,), jnp.float32),
            pltpu.VMEM((LANES,), jnp.float32),
            pltpu.SemaphoreType.DMA,
            pltpu.SemaphoreType.DMA,
            pltpu.SemaphoreType.DMA,
            pltpu.SemaphoreType.DMA,
            pltpu.SemaphoreType.DMA,
        ],
    )
    return run(xT, wT, bias16).reshape(BATCH, 1)
